# Initial kernel scaffold; baseline (speedup 1.0000x reference)
import jax
import jax.numpy as jnp
from jax.experimental import pallas as pl


def kernel(x, weight):
    raise NotImplementedError("write your pallas kernel here")


# speedup vs baseline: 1.5364x; 1.5364x over previous
import functools

import jax
import jax.numpy as jnp
from jax import lax
from jax.experimental import pallas as pl
from jax.experimental.pallas import tpu as pltpu
from jax.experimental.pallas import tpu_sc as plsc

B = 4
S = 4096
HIDDEN = 1024

NC = 2
NS = 16
NW = NC * NS

TOTAL = B * S
B_PER_W = TOTAL // NW
CHUNK = 64
N_CHUNKS = B_PER_W // CHUNK


def _gather_body(idx_hbm, table_hbm, out_hbm, idx_v, rows_v, sem):
    wid = lax.axis_index("s") * NC + lax.axis_index("c")
    base = wid * B_PER_W
    pltpu.sync_copy(idx_hbm.at[pl.ds(base, B_PER_W)], idx_v)
    for i in range(N_CHUNKS):
        pltpu.async_copy(
            table_hbm.at[idx_v.at[pl.ds(i * CHUNK, CHUNK)]],
            rows_v,
            sem,
        ).wait()
        pltpu.sync_copy(rows_v, out_hbm.at[pl.ds(base + i * CHUNK, CHUNK)])


@jax.jit
def _embed(x_flat, weight):
    mesh = plsc.VectorSubcoreMesh(core_axis_name="c", subcore_axis_name="s")
    run = functools.partial(
        pl.kernel,
        mesh=mesh,
        out_type=jax.ShapeDtypeStruct((TOTAL, HIDDEN), jnp.float32),
        scratch_types=[
            pltpu.VMEM((B_PER_W,), jnp.int32),
            pltpu.VMEM((CHUNK, HIDDEN), jnp.float32),
            pltpu.SemaphoreType.DMA,
        ],
    )(_gather_body)
    return run(x_flat, weight)


def kernel(x, weight):
    out = _embed(x.reshape(TOTAL), weight)
    return out.reshape(B, S, HIDDEN)


# speedup vs baseline: 1.6572x; 1.0786x over previous
import functools

import jax
import jax.numpy as jnp
from jax import lax
from jax.experimental import pallas as pl
from jax.experimental.pallas import tpu as pltpu
from jax.experimental.pallas import tpu_sc as plsc

B = 4
S = 4096
HIDDEN = 1024

NC = 2
NS = 16
NW = NC * NS

TOTAL = B * S
B_PER_W = TOTAL // NW
CHUNK = 32
N_CHUNKS = B_PER_W // CHUNK
NBUF = 3


def _gather_body(idx_hbm, table_hbm, out_hbm, idx_v, rows_v, *sems):
    gsems, ssems = sems[:NBUF], sems[NBUF:]
    wid = lax.axis_index("s") * NC + lax.axis_index("c")
    base = wid * B_PER_W
    pltpu.sync_copy(idx_hbm.at[pl.ds(base, B_PER_W)], idx_v)

    def gather(i):
        return pltpu.async_copy(
            table_hbm.at[idx_v.at[pl.ds(i * CHUNK, CHUNK)]],
            rows_v.at[i % NBUF],
            gsems[i % NBUF],
        )

    def scatter(i):
        return pltpu.async_copy(
            rows_v.at[i % NBUF],
            out_hbm.at[pl.ds(base + i * CHUNK, CHUNK)],
            ssems[i % NBUF],
        )

    g = [None] * NBUF
    s = [None] * NBUF
    for i in range(NBUF - 1):
        g[i] = gather(i)
    for i in range(N_CHUNKS):
        b = i % NBUF
        nxt = i + NBUF - 1
        if nxt < N_CHUNKS:
            bn = nxt % NBUF
            if s[bn] is not None:
                s[bn].wait()
            g[bn] = gather(nxt)
        g[b].wait()
        s[b] = scatter(i)
    for i in range(max(0, N_CHUNKS - NBUF), N_CHUNKS):
        s[i % NBUF].wait()


@jax.jit
def _embed(x_flat, weight):
    mesh = plsc.VectorSubcoreMesh(core_axis_name="c", subcore_axis_name="s")
    run = functools.partial(
        pl.kernel,
        mesh=mesh,
        out_type=jax.ShapeDtypeStruct((TOTAL, HIDDEN), jnp.float32),
        scratch_types=[
            pltpu.VMEM((B_PER_W,), jnp.int32),
            pltpu.VMEM((NBUF, CHUNK, HIDDEN), jnp.float32),
        ] + [pltpu.SemaphoreType.DMA] * (2 * NBUF),
    )(_gather_body)
    return run(x_flat, weight)


def kernel(x, weight):
    out = _embed(x.reshape(TOTAL), weight)
    return out.reshape(B, S, HIDDEN)


# speedup vs baseline: 1.6638x; 1.0040x over previous
import functools

import jax
import jax.numpy as jnp
from jax import lax
from jax.experimental import pallas as pl
from jax.experimental.pallas import tpu as pltpu
from jax.experimental.pallas import tpu_sc as plsc

B = 4
S = 4096
HIDDEN = 1024

NC = 2
NS = 16
NW = NC * NS

TOTAL = B * S
B_PER_W = TOTAL // NW
CHUNK = 16
N_CHUNKS = B_PER_W // CHUNK
NBUF = 6


def _gather_body(idx_hbm, table_hbm, out_hbm, idx_v, rows_v, *sems):
    gsems, ssems = sems[:NBUF], sems[NBUF:]
    wid = lax.axis_index("s") * NC + lax.axis_index("c")
    base = wid * B_PER_W
    pltpu.sync_copy(idx_hbm.at[pl.ds(base, B_PER_W)], idx_v)

    def gather(i):
        return pltpu.async_copy(
            table_hbm.at[idx_v.at[pl.ds(i * CHUNK, CHUNK)]],
            rows_v.at[i % NBUF],
            gsems[i % NBUF],
        )

    def scatter(i):
        return pltpu.async_copy(
            rows_v.at[i % NBUF],
            out_hbm.at[pl.ds(base + i * CHUNK, CHUNK)],
            ssems[i % NBUF],
        )

    g = [None] * NBUF
    s = [None] * NBUF
    for i in range(NBUF - 1):
        g[i] = gather(i)
    for i in range(N_CHUNKS):
        b = i % NBUF
        nxt = i + NBUF - 1
        if nxt < N_CHUNKS:
            bn = nxt % NBUF
            if s[bn] is not None:
                s[bn].wait()
            g[bn] = gather(nxt)
        g[b].wait()
        s[b] = scatter(i)
    for i in range(max(0, N_CHUNKS - NBUF), N_CHUNKS):
        s[i % NBUF].wait()


@jax.jit
def _embed(x_flat, weight):
    mesh = plsc.VectorSubcoreMesh(core_axis_name="c", subcore_axis_name="s")
    run = functools.partial(
        pl.kernel,
        mesh=mesh,
        out_type=jax.ShapeDtypeStruct((TOTAL, HIDDEN), jnp.float32),
        scratch_types=[
            pltpu.VMEM((B_PER_W,), jnp.int32),
            pltpu.VMEM((NBUF, CHUNK, HIDDEN), jnp.float32),
        ] + [pltpu.SemaphoreType.DMA] * (2 * NBUF),
    )(_gather_body)
    return run(x_flat, weight)


def kernel(x, weight):
    out = _embed(x.reshape(TOTAL), weight)
    return out.reshape(B, S, HIDDEN)
